# Initial kernel scaffold; baseline (speedup 1.0000x reference)
import jax
import jax.numpy as jnp
from jax.experimental import pallas as pl


def kernel(x, edge_index, W_self0, W_neigh0, b0, W_self1, W_neigh1, b1, W_self2, W_neigh2, b2):
    raise NotImplementedError("write your pallas kernel here")


# speedup vs baseline: 1.5986x; 1.5986x over previous
import jax
import jax.numpy as jnp
from jax import lax
from jax.experimental import pallas as pl
from jax.experimental.pallas import tpu as pltpu
from jax.experimental.pallas import tpu_sc as plsc

N = 10000
E = 160000
F = 256
NC = 2
NS = 16
NW = NC * NS
FPT = F // NW
FPP = 4
CH = 2000
EP = 5008
EPAD = NW * EP

_SC_PARAMS = pltpu.CompilerParams(needs_layout_passes=False)


def _make_seg_T(with_deg):
    mesh = plsc.VectorSubcoreMesh(core_axis_name="c", subcore_axis_name="s")
    out_type = [jax.ShapeDtypeStruct((F, 1, N), jnp.float32)]
    if with_deg:
        out_type.append(jax.ShapeDtypeStruct((NW, 1, N), jnp.float32))
    scratch = (
        [pltpu.VMEM((1, N), jnp.float32) for _ in range(FPP)]
        + [pltpu.VMEM((1, N), jnp.float32) for _ in range(FPP)]
        + [pltpu.VMEM((CH,), jnp.int32),
           pltpu.VMEM((CH,), jnp.int32)]
    )
    if with_deg:
        scratch += [pltpu.VMEM((EP,), jnp.int32),
                    pltpu.VMEM((1, N), jnp.float32)]

    def body(hT, src, dst, dstp, zrow, *rest):
        it = iter(rest)
        aggT = next(it)
        degp = next(it) if with_deg else None
        hrows = [next(it) for _ in range(FPP)]
        accs = [next(it) for _ in range(FPP)]
        srcv = next(it)
        dstv = next(it)
        if with_deg:
            degbuf = next(it)
            degacc = next(it)
        c = lax.axis_index("c")
        s = lax.axis_index("s")
        w = c * NS + s
        z16 = jnp.zeros((16,), jnp.int32)

        if with_deg:
            pltpu.sync_copy(dstp.at[pl.ds(w * EP, EP)], degbuf)
            pltpu.sync_copy(zrow, degacc)
            ones16 = jnp.ones((16,), jnp.float32)
            nj = jnp.where(w == NW - 1, (E - (NW - 1) * EP) // 16, EP // 16)

            def dbody(j, carry):
                d16 = degbuf[pl.ds(j * 16, 16)]
                plsc.addupdate_scatter(degacc, [z16, d16], ones16)
                return carry

            lax.fori_loop(0, nj, dbody, 0)
            pltpu.sync_copy(degacc, degp.at[w])

        for p in range(FPT // FPP):
            fbase = w * FPT + p * FPP
            for k in range(FPP):
                pltpu.sync_copy(zrow, accs[k])
                pltpu.sync_copy(hT.at[fbase + k], hrows[k])

            def gbody(g, carry):
                eo = g * CH
                pltpu.sync_copy(src.at[pl.ds(eo, CH)], srcv)
                pltpu.sync_copy(dst.at[pl.ds(eo, CH)], dstv)

                def jbody(j, carry2):
                    s16 = srcv[pl.ds(j * 16, 16)]
                    d16 = dstv[pl.ds(j * 16, 16)]
                    for k in range(FPP):
                        v = plsc.load_gather(hrows[k], [z16, s16])
                        plsc.addupdate_scatter(accs[k], [z16, d16], v)
                    return carry2

                lax.fori_loop(0, CH // 16, jbody, 0)
                return carry

            lax.fori_loop(0, E // CH, gbody, 0)
            for k in range(FPP):
                pltpu.sync_copy(accs[k], aggT.at[fbase + k])

    return pl.kernel(body, out_type=out_type, mesh=mesh,
                     compiler_params=_SC_PARAMS, scratch_types=scratch)


_seg_T_deg = _make_seg_T(True)
_seg_T = _make_seg_T(False)


def _dot(a, b):
    return lax.dot_general(a, b, (((1,), (0,)), ((), ())),
                           precision=lax.Precision.HIGHEST,
                           preferred_element_type=jnp.float32)


def _tc_transpose_in(x):
    def body(x_ref, o_ref):
        o_ref[...] = jnp.transpose(x_ref[...]).reshape(F, 1, N)

    return pl.pallas_call(
        body,
        out_shape=jax.ShapeDtypeStruct((F, 1, N), jnp.float32),
    )(x)


def _tc_layer_T(hT, aggT, degp, WsT, WnT, bcol, relu):
    do = WsT.shape[0]

    def body(h_ref, a_ref, d_ref, ws_ref, wn_ref, b_ref, o_ref):
        h = h_ref[...].reshape(F, N)
        a = a_ref[...].reshape(F, N)
        deg = jnp.sum(d_ref[...].reshape(NW, N), axis=0, keepdims=True)
        hn = a * (1.0 / jnp.maximum(deg, 1.0))
        res = _dot(ws_ref[...], h) + _dot(wn_ref[...], hn) + b_ref[...]
        if relu:
            res = jnp.maximum(res, 0.0)
        o_ref[...] = res.reshape(do, 1, N)

    return pl.pallas_call(
        body,
        out_shape=jax.ShapeDtypeStruct((do, 1, N), jnp.float32),
    )(hT, aggT, degp, WsT, WnT, bcol)


def _tc_layer_out(hT, aggT, degp, WsT, WnT, bcol):
    do = WsT.shape[0]

    def body(h_ref, a_ref, d_ref, ws_ref, wn_ref, b_ref, o_ref):
        h = h_ref[...].reshape(F, N)
        a = a_ref[...].reshape(F, N)
        deg = jnp.sum(d_ref[...].reshape(NW, N), axis=0, keepdims=True)
        hn = a * (1.0 / jnp.maximum(deg, 1.0))
        res = _dot(ws_ref[...], h) + _dot(wn_ref[...], hn) + b_ref[...]
        o_ref[...] = jnp.transpose(res)

    return pl.pallas_call(
        body,
        out_shape=jax.ShapeDtypeStruct((N, do), jnp.float32),
    )(hT, aggT, degp, WsT, WnT, bcol)


def kernel(x, edge_index, W_self0, W_neigh0, b0, W_self1, W_neigh1, b1,
           W_self2, W_neigh2, b2):
    src = edge_index[0].astype(jnp.int32)
    dst = edge_index[1].astype(jnp.int32)
    dstp = jnp.pad(dst, (0, EPAD - E))
    zrow = jnp.zeros((1, N), jnp.float32)

    xT = _tc_transpose_in(x)
    agg0, degp = _seg_T_deg(xT, src, dst, dstp, zrow)
    h1T = _tc_layer_T(xT, agg0, degp, W_self0.T, W_neigh0.T,
                      b0.reshape(-1, 1), relu=True)
    (agg1,) = _seg_T(h1T, src, dst, dstp, zrow)
    h2T = _tc_layer_T(h1T, agg1, degp, W_self1.T, W_neigh1.T,
                      b1.reshape(-1, 1), relu=True)
    (agg2,) = _seg_T(h2T, src, dst, dstp, zrow)
    return _tc_layer_out(h2T, agg2, degp, W_self2.T, W_neigh2.T,
                         b2.reshape(-1, 1))


# speedup vs baseline: 2.1783x; 1.3626x over previous
import jax
import jax.numpy as jnp
from jax import lax
from jax.experimental import pallas as pl
from jax.experimental.pallas import tpu as pltpu
from jax.experimental.pallas import tpu_sc as plsc

N = 10000
E = 160000
F = 256
NC = 2
NS = 16
NW = NC * NS
FPT = F // NW
FPP = 4
CH = 2000
EP = 5008
EPAD = NW * EP

_SC_PARAMS = pltpu.CompilerParams(needs_layout_passes=False)


def _make_seg_T(with_deg):
    mesh = plsc.VectorSubcoreMesh(core_axis_name="c", subcore_axis_name="s")
    out_type = [jax.ShapeDtypeStruct((F, 1, N), jnp.float32)]
    if with_deg:
        out_type.append(jax.ShapeDtypeStruct((NW, 1, N), jnp.float32))
    scratch = (
        [pltpu.VMEM((1, N), jnp.float32) for _ in range(FPP)]
        + [pltpu.VMEM((1, N), jnp.float32) for _ in range(FPP)]
        + [pltpu.VMEM((CH,), jnp.int32) for _ in range(2)]
        + [pltpu.VMEM((CH,), jnp.int32) for _ in range(2)]
        + [pltpu.SemaphoreType.DMA for _ in range(2)]
    )
    if with_deg:
        scratch += [pltpu.VMEM((EP,), jnp.int32),
                    pltpu.VMEM((1, N), jnp.float32)]

    def body(hT, src, dst, dstp, zrow, *rest):
        it = iter(rest)
        aggT = next(it)
        degp = next(it) if with_deg else None
        hrows = [next(it) for _ in range(FPP)]
        accs = [next(it) for _ in range(FPP)]
        sbuf = [next(it) for _ in range(2)]
        dbuf = [next(it) for _ in range(2)]
        isem = [next(it) for _ in range(2)]
        if with_deg:
            degbuf = next(it)
            degacc = next(it)
        c = lax.axis_index("c")
        s = lax.axis_index("s")
        w = c * NS + s
        z16 = jnp.zeros((16,), jnp.int32)

        if with_deg:
            pltpu.sync_copy(dstp.at[pl.ds(w * EP, EP)], degbuf)
            pltpu.sync_copy(zrow, degacc)
            ones16 = jnp.ones((16,), jnp.float32)
            nj = jnp.where(w == NW - 1, (E - (NW - 1) * EP) // 16, EP // 16)

            def dbody(j, carry):
                d16 = degbuf[pl.ds(j * 16, 16)]
                plsc.addupdate_scatter(degacc, [z16, d16], ones16)
                return carry

            lax.fori_loop(0, nj, dbody, 0)
            pltpu.sync_copy(degacc, degp.at[w])

        def fire(b, g):
            eo = g * CH
            pltpu.async_copy(src.at[pl.ds(eo, CH)], sbuf[b], isem[b])
            pltpu.async_copy(dst.at[pl.ds(eo, CH)], dbuf[b], isem[b])

        def drain(b, g):
            eo = g * CH
            pltpu.make_async_copy(src.at[pl.ds(eo, CH)], sbuf[b],
                                  isem[b]).wait()
            pltpu.make_async_copy(dst.at[pl.ds(eo, CH)], dbuf[b],
                                  isem[b]).wait()

        NPAIR = (E // CH) // 2

        for p in range(FPT // FPP):
            fbase = w * FPT + p * FPP
            for k in range(FPP):
                pltpu.sync_copy(zrow, accs[k])
                pltpu.sync_copy(hT.at[fbase + k], hrows[k])

            def compute(b):
                def jbody(j, carry2):
                    s16 = sbuf[b][pl.ds(j * 16, 16)]
                    d16 = dbuf[b][pl.ds(j * 16, 16)]
                    for k in range(FPP):
                        v = plsc.load_gather(hrows[k], [z16, s16])
                        plsc.addupdate_scatter(accs[k], [z16, d16], v)
                    return carry2

                lax.fori_loop(0, CH // 16, jbody, 0, unroll=4)

            fire(0, 0)

            def gbody(gg, carry):
                g0 = 2 * gg
                fire(1, g0 + 1)
                drain(0, g0)
                compute(0)

                @pl.when(gg < NPAIR - 1)
                def _():
                    fire(0, g0 + 2)

                drain(1, g0 + 1)
                compute(1)
                return carry

            lax.fori_loop(0, NPAIR, gbody, 0)
            for k in range(FPP):
                pltpu.sync_copy(accs[k], aggT.at[fbase + k])

    return pl.kernel(body, out_type=out_type, mesh=mesh,
                     compiler_params=_SC_PARAMS, scratch_types=scratch)


_seg_T_deg = _make_seg_T(True)
_seg_T = _make_seg_T(False)


def _dot(a, b):
    return lax.dot_general(a, b, (((1,), (0,)), ((), ())),
                           precision=lax.Precision.HIGHEST,
                           preferred_element_type=jnp.float32)


def _tc_transpose_in(x):
    def body(x_ref, o_ref):
        o_ref[...] = jnp.transpose(x_ref[...]).reshape(F, 1, N)

    return pl.pallas_call(
        body,
        out_shape=jax.ShapeDtypeStruct((F, 1, N), jnp.float32),
    )(x)


def _tc_layer_T(hT, aggT, degp, WsT, WnT, bcol, relu):
    do = WsT.shape[0]

    def body(h_ref, a_ref, d_ref, ws_ref, wn_ref, b_ref, o_ref):
        h = h_ref[...].reshape(F, N)
        a = a_ref[...].reshape(F, N)
        deg = jnp.sum(d_ref[...].reshape(NW, N), axis=0, keepdims=True)
        hn = a * (1.0 / jnp.maximum(deg, 1.0))
        res = _dot(ws_ref[...], h) + _dot(wn_ref[...], hn) + b_ref[...]
        if relu:
            res = jnp.maximum(res, 0.0)
        o_ref[...] = res.reshape(do, 1, N)

    return pl.pallas_call(
        body,
        out_shape=jax.ShapeDtypeStruct((do, 1, N), jnp.float32),
    )(hT, aggT, degp, WsT, WnT, bcol)


def _tc_layer_out(hT, aggT, degp, WsT, WnT, bcol):
    do = WsT.shape[0]

    def body(h_ref, a_ref, d_ref, ws_ref, wn_ref, b_ref, o_ref):
        h = h_ref[...].reshape(F, N)
        a = a_ref[...].reshape(F, N)
        deg = jnp.sum(d_ref[...].reshape(NW, N), axis=0, keepdims=True)
        hn = a * (1.0 / jnp.maximum(deg, 1.0))
        res = _dot(ws_ref[...], h) + _dot(wn_ref[...], hn) + b_ref[...]
        o_ref[...] = jnp.transpose(res)

    return pl.pallas_call(
        body,
        out_shape=jax.ShapeDtypeStruct((N, do), jnp.float32),
    )(hT, aggT, degp, WsT, WnT, bcol)


def kernel(x, edge_index, W_self0, W_neigh0, b0, W_self1, W_neigh1, b1,
           W_self2, W_neigh2, b2):
    src = edge_index[0].astype(jnp.int32)
    dst = edge_index[1].astype(jnp.int32)
    dstp = jnp.pad(dst, (0, EPAD - E))
    zrow = jnp.zeros((1, N), jnp.float32)

    xT = _tc_transpose_in(x)
    agg0, degp = _seg_T_deg(xT, src, dst, dstp, zrow)
    h1T = _tc_layer_T(xT, agg0, degp, W_self0.T, W_neigh0.T,
                      b0.reshape(-1, 1), relu=True)
    (agg1,) = _seg_T(h1T, src, dst, dstp, zrow)
    h2T = _tc_layer_T(h1T, agg1, degp, W_self1.T, W_neigh1.T,
                      b1.reshape(-1, 1), relu=True)
    (agg2,) = _seg_T(h2T, src, dst, dstp, zrow)
    return _tc_layer_out(h2T, agg2, degp, W_self2.T, W_neigh2.T,
                         b2.reshape(-1, 1))


# speedup vs baseline: 2.1808x; 1.0012x over previous
import jax
import jax.numpy as jnp
from jax import lax
from jax.experimental import pallas as pl
from jax.experimental.pallas import tpu as pltpu
from jax.experimental.pallas import tpu_sc as plsc

N = 10000
E = 160000
F = 256
NC = 2
NS = 16
NW = NC * NS
FPT = F // NW
FPP = 4
CH = 2000
EP = 5008
EPAD = NW * EP

_SC_PARAMS = pltpu.CompilerParams(needs_layout_passes=False)


def _make_seg_T(with_deg):
    mesh = plsc.VectorSubcoreMesh(core_axis_name="c", subcore_axis_name="s")
    out_type = [jax.ShapeDtypeStruct((F, 1, N), jnp.float32)]
    if with_deg:
        out_type.append(jax.ShapeDtypeStruct((NW, 1, N), jnp.float32))
    scratch = (
        [pltpu.VMEM((1, N), jnp.float32) for _ in range(FPP)]
        + [pltpu.VMEM((1, N), jnp.float32) for _ in range(FPP)]
        + [pltpu.VMEM((CH,), jnp.int32) for _ in range(2)]
        + [pltpu.VMEM((CH,), jnp.int32) for _ in range(2)]
        + [pltpu.SemaphoreType.DMA for _ in range(2)]
    )
    if with_deg:
        scratch += [pltpu.VMEM((EP,), jnp.int32),
                    pltpu.VMEM((1, N), jnp.float32)]

    def body(hT, src, dst, dstp, zrow, *rest):
        it = iter(rest)
        aggT = next(it)
        degp = next(it) if with_deg else None
        hrows = [next(it) for _ in range(FPP)]
        accs = [next(it) for _ in range(FPP)]
        sbuf = [next(it) for _ in range(2)]
        dbuf = [next(it) for _ in range(2)]
        isem = [next(it) for _ in range(2)]
        if with_deg:
            degbuf = next(it)
            degacc = next(it)
        c = lax.axis_index("c")
        s = lax.axis_index("s")
        w = c * NS + s
        z16 = jnp.zeros((16,), jnp.int32)

        if with_deg:
            pltpu.sync_copy(dstp.at[pl.ds(w * EP, EP)], degbuf)
            pltpu.sync_copy(zrow, degacc)
            ones16 = jnp.ones((16,), jnp.float32)
            nj = jnp.where(w == NW - 1, (E - (NW - 1) * EP) // 16, EP // 16)

            def dbody(j, carry):
                d16 = degbuf[pl.ds(j * 16, 16)]
                plsc.addupdate_scatter(degacc, [z16, d16], ones16)
                return carry

            lax.fori_loop(0, nj, dbody, 0)
            pltpu.sync_copy(degacc, degp.at[w])

        def fire(b, g):
            eo = g * CH
            pltpu.async_copy(src.at[pl.ds(eo, CH)], sbuf[b], isem[b])
            pltpu.async_copy(dst.at[pl.ds(eo, CH)], dbuf[b], isem[b])

        def drain(b, g):
            eo = g * CH
            pltpu.make_async_copy(src.at[pl.ds(eo, CH)], sbuf[b],
                                  isem[b]).wait()
            pltpu.make_async_copy(dst.at[pl.ds(eo, CH)], dbuf[b],
                                  isem[b]).wait()

        NPAIR = (E // CH) // 2

        for p in range(FPT // FPP):
            fbase = w * FPT + p * FPP
            for k in range(FPP):
                pltpu.sync_copy(zrow, accs[k])
                pltpu.sync_copy(hT.at[fbase + k], hrows[k])

            def compute(b):
                def jbody(j, carry2):
                    s16 = sbuf[b][pl.ds(j * 16, 16)]
                    d16 = dbuf[b][pl.ds(j * 16, 16)]
                    for k in range(FPP):
                        v = plsc.load_gather(hrows[k], [z16, s16])
                        plsc.addupdate_scatter(accs[k], [z16, d16], v)
                    return carry2

                lax.fori_loop(0, CH // 16, jbody, 0, unroll=8)

            fire(0, 0)

            def gbody(gg, carry):
                g0 = 2 * gg
                fire(1, g0 + 1)
                drain(0, g0)
                compute(0)

                @pl.when(gg < NPAIR - 1)
                def _():
                    fire(0, g0 + 2)

                drain(1, g0 + 1)
                compute(1)
                return carry

            lax.fori_loop(0, NPAIR, gbody, 0)
            for k in range(FPP):
                pltpu.sync_copy(accs[k], aggT.at[fbase + k])

    return pl.kernel(body, out_type=out_type, mesh=mesh,
                     compiler_params=_SC_PARAMS, scratch_types=scratch)


_seg_T_deg = _make_seg_T(True)
_seg_T = _make_seg_T(False)


def _dot(a, b):
    return lax.dot_general(a, b, (((1,), (0,)), ((), ())),
                           precision=lax.Precision.HIGHEST,
                           preferred_element_type=jnp.float32)


def _tc_transpose_in(x):
    def body(x_ref, o_ref):
        o_ref[...] = jnp.transpose(x_ref[...]).reshape(F, 1, N)

    return pl.pallas_call(
        body,
        out_shape=jax.ShapeDtypeStruct((F, 1, N), jnp.float32),
    )(x)


def _tc_layer_T(hT, aggT, degp, WsT, WnT, bcol, relu):
    do = WsT.shape[0]

    def body(h_ref, a_ref, d_ref, ws_ref, wn_ref, b_ref, o_ref):
        h = h_ref[...].reshape(F, N)
        a = a_ref[...].reshape(F, N)
        deg = jnp.sum(d_ref[...].reshape(NW, N), axis=0, keepdims=True)
        hn = a * (1.0 / jnp.maximum(deg, 1.0))
        res = _dot(ws_ref[...], h) + _dot(wn_ref[...], hn) + b_ref[...]
        if relu:
            res = jnp.maximum(res, 0.0)
        o_ref[...] = res.reshape(do, 1, N)

    return pl.pallas_call(
        body,
        out_shape=jax.ShapeDtypeStruct((do, 1, N), jnp.float32),
    )(hT, aggT, degp, WsT, WnT, bcol)


def _tc_layer_out(hT, aggT, degp, WsT, WnT, bcol):
    do = WsT.shape[0]

    def body(h_ref, a_ref, d_ref, ws_ref, wn_ref, b_ref, o_ref):
        h = h_ref[...].reshape(F, N)
        a = a_ref[...].reshape(F, N)
        deg = jnp.sum(d_ref[...].reshape(NW, N), axis=0, keepdims=True)
        hn = a * (1.0 / jnp.maximum(deg, 1.0))
        res = _dot(ws_ref[...], h) + _dot(wn_ref[...], hn) + b_ref[...]
        o_ref[...] = jnp.transpose(res)

    return pl.pallas_call(
        body,
        out_shape=jax.ShapeDtypeStruct((N, do), jnp.float32),
    )(hT, aggT, degp, WsT, WnT, bcol)


def kernel(x, edge_index, W_self0, W_neigh0, b0, W_self1, W_neigh1, b1,
           W_self2, W_neigh2, b2):
    src = edge_index[0].astype(jnp.int32)
    dst = edge_index[1].astype(jnp.int32)
    dstp = jnp.pad(dst, (0, EPAD - E))
    zrow = jnp.zeros((1, N), jnp.float32)

    xT = _tc_transpose_in(x)
    agg0, degp = _seg_T_deg(xT, src, dst, dstp, zrow)
    h1T = _tc_layer_T(xT, agg0, degp, W_self0.T, W_neigh0.T,
                      b0.reshape(-1, 1), relu=True)
    (agg1,) = _seg_T(h1T, src, dst, dstp, zrow)
    h2T = _tc_layer_T(h1T, agg1, degp, W_self1.T, W_neigh1.T,
                      b1.reshape(-1, 1), relu=True)
    (agg2,) = _seg_T(h2T, src, dst, dstp, zrow)
    return _tc_layer_out(h2T, agg2, degp, W_self2.T, W_neigh2.T,
                         b2.reshape(-1, 1))
